# baseline (device time: 157440 ns/iter reference)
import jax
import jax.numpy as jnp
from jax import lax
from jax.experimental import pallas as pl
from jax.experimental.pallas import tpu as pltpu

N_DEV = 8
SQ = 512
SKV = 2048
D_MODEL = 1024
HQ_PER = 8
DH = 128
SCALE = 0.08838834764831843


def _ring_allreduce(partial):

    def body(p_ref, out_ref, comm_ref, send_sems, recv_sems):
        my = lax.axis_index("i")
        left = lax.rem(my + N_DEV - 1, N_DEV)
        right = lax.rem(my + 1, N_DEV)

        barrier_sem = pltpu.get_barrier_semaphore()
        for nbr in (left, right):
            pl.semaphore_signal(
                barrier_sem, inc=1,
                device_id=(nbr,), device_id_type=pl.DeviceIdType.MESH,
            )
        pl.semaphore_wait(barrier_sem, 2)

        comm_ref[0] = p_ref[...]
        out_ref[...] = p_ref[...].astype(jnp.float32)

        for h in range(N_DEV - 1):
            rdma = pltpu.make_async_remote_copy(
                src_ref=comm_ref.at[h],
                dst_ref=comm_ref.at[h + 1],
                send_sem=send_sems.at[h],
                recv_sem=recv_sems.at[h],
                device_id=(right,),
                device_id_type=pl.DeviceIdType.MESH,
            )
            rdma.start()
            rdma.wait()
            out_ref[...] += comm_ref[h + 1].astype(jnp.float32)

    return pl.pallas_call(
        body,
        out_shape=jax.ShapeDtypeStruct((SQ, D_MODEL), jnp.float32),
        in_specs=[pl.BlockSpec(memory_space=pltpu.VMEM)],
        out_specs=pl.BlockSpec(memory_space=pltpu.VMEM),
        scratch_shapes=[
            pltpu.VMEM((N_DEV, SQ, D_MODEL), jnp.bfloat16),
            pltpu.SemaphoreType.DMA((N_DEV - 1,)),
            pltpu.SemaphoreType.DMA((N_DEV - 1,)),
        ],
        compiler_params=pltpu.CompilerParams(collective_id=0),
    )(partial)


def kernel(x, Wq, Wo, K_ext, V_ext):
    my = lax.axis_index("i")

    xb = x[0].astype(jnp.bfloat16)
    Q = jnp.dot(xb, Wq.astype(jnp.bfloat16),
                preferred_element_type=jnp.float32)
    Q = Q.astype(jnp.bfloat16).reshape(SQ, HQ_PER, DH)

    K = lax.dynamic_slice_in_dim(K_ext[0], my * HQ_PER, HQ_PER, axis=1)
    V = lax.dynamic_slice_in_dim(V_ext[0], my * HQ_PER, HQ_PER, axis=1)
    K = K.astype(jnp.bfloat16)
    V = V.astype(jnp.bfloat16)

    s = jnp.einsum("qhd,khd->hqk", Q, K,
                   preferred_element_type=jnp.float32) * SCALE
    p = jax.nn.softmax(s, axis=-1).astype(jnp.bfloat16)
    o = jnp.einsum("hqk,khd->qhd", p, V,
                   preferred_element_type=jnp.float32)
    o = o.reshape(SQ, HQ_PER * DH).astype(jnp.bfloat16)

    partial = jnp.dot(o, Wo.astype(jnp.bfloat16),
                      preferred_element_type=jnp.float32)
    partial = partial.astype(jnp.bfloat16)

    out = _ring_allreduce(partial)
    return out.reshape(1, SQ, D_MODEL)


# device time: 79243 ns/iter; 1.9868x vs baseline; 1.9868x over previous
import jax
import jax.numpy as jnp
from jax import lax
from jax.experimental import pallas as pl
from jax.experimental.pallas import tpu as pltpu

N_DEV = 8
SQ = 512
SKV = 2048
D_MODEL = 1024
HQ_PER = 8
DH = 128
SCALE = 0.08838834764831843


def _attention(xb, Wq, K, V):

    def body(x_ref, wq_ref, k_ref, v_ref, o_ref):
        q = jnp.dot(x_ref[...], wq_ref[...],
                    preferred_element_type=jnp.float32)
        q = q.astype(jnp.bfloat16)
        k = k_ref[0]
        v = v_ref[0]
        s = lax.dot_general(q, k, (((1,), (1,)), ((), ())),
                            preferred_element_type=jnp.float32) * SCALE
        m = jnp.max(s, axis=-1, keepdims=True)
        p = jnp.exp(s - m)
        l = jnp.sum(p, axis=-1, keepdims=True)
        o = jnp.dot(p.astype(jnp.bfloat16), v,
                    preferred_element_type=jnp.float32) / l
        o_ref[...] = o.astype(jnp.bfloat16)

    return pl.pallas_call(
        body,
        grid=(HQ_PER,),
        in_specs=[
            pl.BlockSpec((SQ, D_MODEL), lambda h: (0, 0)),
            pl.BlockSpec((D_MODEL, DH), lambda h: (0, h)),
            pl.BlockSpec((1, SKV, DH), lambda h: (h, 0, 0)),
            pl.BlockSpec((1, SKV, DH), lambda h: (h, 0, 0)),
        ],
        out_specs=pl.BlockSpec((SQ, DH), lambda h: (0, h)),
        out_shape=jax.ShapeDtypeStruct((SQ, HQ_PER * DH), jnp.bfloat16),
    )(xb, Wq, K, V)


def _project_allreduce(o, Wo):

    def body(o_ref, wo_ref, out_ref, stage_ref, recv_ref,
             send_sems, recv_sems):
        my = lax.axis_index("i")
        r1 = (my >> 1) & 1
        r2 = my & 1
        r3 = (my >> 2) & 1
        partners = (my ^ 3, my ^ 1, my ^ 4)

        barrier_sem = pltpu.get_barrier_semaphore()
        for nbr in partners:
            pl.semaphore_signal(
                barrier_sem, inc=1,
                device_id=(nbr,), device_id_type=pl.DeviceIdType.MESH,
            )
        pl.semaphore_wait(barrier_sem, 3)

        out_ref[...] = jnp.dot(o_ref[...], wo_ref[...],
                               preferred_element_type=jnp.float32)

        kept1_lo = r1 * 256
        sent1_lo = (1 - r1) * 256
        kept2_lo = kept1_lo + r2 * 128
        sent2_lo = kept1_lo + (1 - r2) * 128
        own_lo = kept2_lo + r3 * 64
        sent3_lo = kept2_lo + (1 - r3) * 64

        rs_rounds = (
            (0, sent1_lo, kept1_lo, 256, 0),
            (1, sent2_lo, kept2_lo, 128, 256),
            (2, sent3_lo, own_lo, 64, 384),
        )
        for idx, s_lo, k_lo, ln, r_off in rs_rounds:
            stage_ref[pl.ds(s_lo, ln), :] = (
                out_ref[pl.ds(s_lo, ln), :].astype(jnp.bfloat16))
            rdma = pltpu.make_async_remote_copy(
                src_ref=stage_ref.at[pl.ds(s_lo, ln), :],
                dst_ref=recv_ref.at[pl.ds(r_off, ln), :],
                send_sem=send_sems.at[idx],
                recv_sem=recv_sems.at[idx],
                device_id=(partners[idx],),
                device_id_type=pl.DeviceIdType.MESH,
            )
            rdma.start()
            rdma.wait()
            out_ref[pl.ds(k_lo, ln), :] += (
                recv_ref[pl.ds(r_off, ln), :].astype(jnp.float32))

        stage_ref[pl.ds(own_lo, 64), :] = (
            out_ref[pl.ds(own_lo, 64), :].astype(jnp.bfloat16))
        ag_rounds = (
            (3, 2, own_lo, 64),
            (4, 1, kept2_lo, 128),
            (5, 0, kept1_lo, 256),
        )
        for idx, p_idx, lo, ln in ag_rounds:
            rdma = pltpu.make_async_remote_copy(
                src_ref=stage_ref.at[pl.ds(lo, ln), :],
                dst_ref=stage_ref.at[pl.ds(lo, ln), :],
                send_sem=send_sems.at[idx],
                recv_sem=recv_sems.at[idx],
                device_id=(partners[p_idx],),
                device_id_type=pl.DeviceIdType.MESH,
            )
            rdma.start()
            rdma.wait()

        out_ref[...] = stage_ref[...].astype(jnp.float32)

    return pl.pallas_call(
        body,
        out_shape=jax.ShapeDtypeStruct((SQ, D_MODEL), jnp.float32),
        in_specs=[
            pl.BlockSpec(memory_space=pltpu.VMEM),
            pl.BlockSpec(memory_space=pltpu.VMEM),
        ],
        out_specs=pl.BlockSpec(memory_space=pltpu.VMEM),
        scratch_shapes=[
            pltpu.VMEM((SQ, D_MODEL), jnp.bfloat16),
            pltpu.VMEM((448, D_MODEL), jnp.bfloat16),
            pltpu.SemaphoreType.DMA((6,)),
            pltpu.SemaphoreType.DMA((6,)),
        ],
        compiler_params=pltpu.CompilerParams(collective_id=0),
    )(o, Wo)


def kernel(x, Wq, Wo, K_ext, V_ext):
    my = lax.axis_index("i")

    xb = x[0].astype(jnp.bfloat16)
    K = lax.dynamic_slice_in_dim(K_ext[0], my * HQ_PER, HQ_PER, axis=1)
    V = lax.dynamic_slice_in_dim(V_ext[0], my * HQ_PER, HQ_PER, axis=1)
    K = K.transpose(1, 0, 2)
    V = V.transpose(1, 0, 2)

    o = _attention(xb, Wq.astype(jnp.bfloat16),
                   K.astype(jnp.bfloat16), V.astype(jnp.bfloat16))
    out = _project_allreduce(o, Wo.astype(jnp.bfloat16))
    return out.reshape(1, SQ, D_MODEL)
